# baseline (speedup 1.0000x reference)
import functools

import jax
import jax.numpy as jnp
from jax import lax
from jax.experimental import pallas as pl
from jax.experimental.pallas import tpu as pltpu
from jax.experimental.pallas import tpu_sc as plsc

N_STATES = 8192
BATCH = 4096
LANES = 16
NC, NS = 2, 16
NW = NC * NS
ROWS_PER_W = N_STATES // NW
CHUNKS = BATCH // LANES


def _sc_body(state_hbm, g_hbm, t_hbm, out_hbm,
             state_v, row_v, g_v, out_v, sem_r, sem_g):
    wid = lax.axis_index("s") * NC + lax.axis_index("c")
    base = wid * ROWS_PER_W
    pltpu.sync_copy(state_hbm, state_v)
    iota = lax.iota(jnp.int32, LANES)

    def per_group(grp, carry):
        def per_row(rr, acc):
            row_id = base + grp * LANES + rr
            cp_r = pltpu.make_async_copy(t_hbm.at[row_id], row_v, sem_r)
            cp_g = pltpu.make_async_copy(g_hbm.at[row_id], g_v, sem_g)
            cp_r.start()
            cp_g.start()
            cp_r.wait()
            cp_g.wait()

            def inner(k, c):
                bv, bj = c
                off = k * LANES
                idx = state_v[pl.ds(off, LANES)]
                val = plsc.load_gather(row_v, [idx]) + g_v[pl.ds(off, LANES)]
                j = off + iota
                upd = val > bv
                return (jnp.where(upd, val, bv), jnp.where(upd, j, bj))

            init = (jnp.full((LANES,), -jnp.inf, jnp.float32),
                    jnp.zeros((LANES,), jnp.int32))
            bv, bj = lax.fori_loop(0, CHUNKS, inner, init)
            m = jnp.max(bv)
            mj = jnp.min(jnp.where(bv == m, bj, jnp.int32(2**30)))
            return jnp.where(iota == rr, mj, acc)

        acc = lax.fori_loop(0, LANES, per_row, jnp.zeros((LANES,), jnp.int32))
        out_v[pl.ds(grp * LANES, LANES)] = acc
        return carry

    lax.fori_loop(0, ROWS_PER_W // LANES, per_group, 0)
    pltpu.sync_copy(out_v, out_hbm.at[pl.ds(base, ROWS_PER_W)])


_sc_call = pl.kernel(
    _sc_body,
    out_type=jax.ShapeDtypeStruct((N_STATES,), jnp.int32),
    mesh=plsc.VectorSubcoreMesh(core_axis_name="c", subcore_axis_name="s",
                                num_cores=NC, num_subcores=NS),
    scratch_types=[
        pltpu.VMEM((BATCH,), jnp.int32),
        pltpu.VMEM((N_STATES,), jnp.float32),
        pltpu.VMEM((BATCH,), jnp.float32),
        pltpu.VMEM((ROWS_PER_W,), jnp.int32),
        pltpu.SemaphoreType.DMA,
        pltpu.SemaphoreType.DMA,
    ],
    compiler_params=pltpu.CompilerParams(use_tc_tiling_on_sc=False,
                                         needs_layout_passes=False),
)


def kernel(state, rng, trans_matrix):
    g = jax.random.gumbel(rng, (N_STATES, BATCH), jnp.float32)
    return _sc_call(state, g, trans_matrix)


# speedup vs baseline: 1.1016x; 1.1016x over previous
import functools

import jax
import jax.numpy as jnp
import numpy as np
from jax import lax
from jax.experimental import pallas as pl
from jax.experimental.pallas import tpu as pltpu
from jax.experimental.pallas import tpu_sc as plsc

N_STATES = 8192
BATCH = 4096
LANES = 16
NC, NS = 2, 16
NW = NC * NS
NB = 8
CHUNK_ROWS = N_STATES // NB
ROWS_PER_W = CHUNK_ROWS // NW
GROUP = 4
CHUNKS16 = BATCH // LANES

_TINY = np.float32(np.finfo(np.float32).tiny)


_ROTS = ((13, 15, 26, 6), (17, 29, 16, 24))


def _threefry_gumbel(k1, k2, p):
    ks2 = k1 ^ k2 ^ jnp.int32(0x1BD11BDA)
    x0 = k1 + jnp.zeros_like(p)
    x1 = p + k2
    inj = ((k2, ks2, 1), (ks2, k1, 2), (k1, k2, 3), (k2, ks2, 4), (ks2, k1, 5))
    for grp in range(5):
        for r in _ROTS[grp % 2]:
            x0 = x0 + x1
            x1 = ((x1 << np.int32(r)) |
                  lax.shift_right_logical(x1, np.int32(32 - r)))
            x1 = x1 ^ x0
        ka, kb, inc = inj[grp]
        x0 = x0 + ka
        x1 = x1 + kb + jnp.int32(inc)
    bits = x0 ^ x1
    fb = lax.shift_right_logical(bits, np.int32(9)) | jnp.int32(0x3F800000)
    f = lax.bitcast_convert_type(fb, jnp.float32) - np.float32(1.0)
    u = jnp.maximum(_TINY, f * (np.float32(1.0) - _TINY) + _TINY)
    return -jnp.log(-jnp.log(u))


BR = 8
JC = 512


def _tc_gumbel_body(row0, kd_ref, g_ref):
    b = pl.program_id(0)
    k1 = kd_ref[0]
    k2 = kd_ref[1]
    iota_r = lax.broadcasted_iota(jnp.int32, (BR, JC), 0)
    iota_c = lax.broadcasted_iota(jnp.int32, (BR, JC), 1)
    tile_iota = iota_r * jnp.int32(BATCH) + iota_c
    base_row = row0 + b * BR

    def jstep(c, _):
        p = (base_row * jnp.int32(BATCH) + c * jnp.int32(JC)) + tile_iota
        g_ref[:, pl.ds(c * JC, JC)] = _threefry_gumbel(k1, k2, p)
        return _

    lax.fori_loop(0, BATCH // JC, jstep, 0)


def _make_tc_gumbel(row0):
    return pl.pallas_call(
        functools.partial(_tc_gumbel_body, row0),
        grid=(CHUNK_ROWS // BR,),
        in_specs=[pl.BlockSpec(memory_space=pltpu.SMEM)],
        out_specs=pl.BlockSpec((BR, BATCH), lambda b: (b, 0)),
        out_shape=jax.ShapeDtypeStruct((CHUNK_ROWS, BATCH), jnp.float32),
    )


def _sc_body(row0, state_hbm, g_hbm, t_hbm, out_hbm,
             state_v, rows_v, g_v, out_v, sem_r, sem_g):
    wid = lax.axis_index("s") * NC + lax.axis_index("c")
    lbase = wid * ROWS_PER_W
    pltpu.sync_copy(state_hbm, state_v)
    iota = lax.iota(jnp.int32, LANES)
    neg_inf = jnp.full((LANES,), -jnp.inf, jnp.float32)
    zero_i = jnp.zeros((LANES,), jnp.int32)
    big = jnp.int32(2 ** 30)

    acc = zero_i
    for grp in range(ROWS_PER_W // GROUP):
        lrow = lbase + grp * GROUP
        cp_r = pltpu.make_async_copy(
            t_hbm.at[pl.ds(row0 + lrow, GROUP), :], rows_v, sem_r)
        cp_g = pltpu.make_async_copy(
            g_hbm.at[pl.ds(lrow, GROUP), :], g_v, sem_g)
        cp_r.start()
        cp_g.start()
        cp_r.wait()
        cp_g.wait()

        def inner(k, c):
            off = k * LANES
            idx = state_v[pl.ds(off, LANES)]
            j = off + iota
            new = []
            for r in range(GROUP):
                bv, bj = c[2 * r], c[2 * r + 1]
                val = (plsc.load_gather(rows_v, [jnp.full((LANES,), r, jnp.int32), idx])
                       + g_v[r, pl.ds(off, LANES)])
                upd = val > bv
                new.append(jnp.where(upd, val, bv))
                new.append(jnp.where(upd, j, bj))
            return tuple(new)

        init = (neg_inf, zero_i) * GROUP
        res = lax.fori_loop(0, CHUNKS16, inner, init)
        for r in range(GROUP):
            bv, bj = res[2 * r], res[2 * r + 1]
            m = jnp.max(bv)
            mj = jnp.min(jnp.where(bv == m, bj, big))
            lane = (grp * GROUP + r) % LANES
            acc = jnp.where(iota == lane, mj, acc)
        if (grp * GROUP + GROUP) % LANES == 0:
            vec = ((grp * GROUP) // LANES) * LANES
            out_v[pl.ds(vec, LANES)] = acc

    pltpu.sync_copy(out_v, out_hbm.at[pl.ds(lbase, ROWS_PER_W)])


def _make_sc_chunk(row0):
    return pl.kernel(
        functools.partial(_sc_body, row0),
        out_type=jax.ShapeDtypeStruct((CHUNK_ROWS,), jnp.int32),
        mesh=plsc.VectorSubcoreMesh(core_axis_name="c", subcore_axis_name="s",
                                    num_cores=NC, num_subcores=NS),
        scratch_types=[
            pltpu.VMEM((BATCH,), jnp.int32),
            pltpu.VMEM((GROUP, N_STATES), jnp.float32),
            pltpu.VMEM((GROUP, BATCH), jnp.float32),
            pltpu.VMEM((ROWS_PER_W,), jnp.int32),
            pltpu.SemaphoreType.DMA,
            pltpu.SemaphoreType.DMA,
        ],
        compiler_params=pltpu.CompilerParams(use_tc_tiling_on_sc=False,
                                             needs_layout_passes=False),
    )


_TC_CALLS = [_make_tc_gumbel(c * CHUNK_ROWS) for c in range(NB)]


@functools.lru_cache(maxsize=None)
def _sc_calls():
    return [_make_sc_chunk(c * CHUNK_ROWS) for c in range(NB)]


def kernel(state, rng, trans_matrix):
    kd = lax.bitcast_convert_type(jax.random.key_data(rng), jnp.int32)
    sc = _sc_calls()
    outs = []
    for c in range(NB):
        g_c = _TC_CALLS[c](kd)
        outs.append(sc[c](state, g_c, trans_matrix))
    return jnp.concatenate(outs)


# speedup vs baseline: 1.2561x; 1.1402x over previous
import functools

import jax
import jax.numpy as jnp
import numpy as np
from jax import lax
from jax.experimental import pallas as pl
from jax.experimental.pallas import tpu as pltpu
from jax.experimental.pallas import tpu_sc as plsc

N_STATES = 8192
BATCH = 4096
LANES = 16
NC, NS = 2, 16
NW = NC * NS
NB = 8
CHUNK_ROWS = N_STATES // NB
ROWS_PER_W = CHUNK_ROWS // NW
GROUP = 8
CHUNKS16 = BATCH // LANES

_TINY = np.float32(np.finfo(np.float32).tiny)


_ROTS = ((13, 15, 26, 6), (17, 29, 16, 24))


def _threefry_gumbel(k1, k2, p):
    ks2 = k1 ^ k2 ^ jnp.int32(0x1BD11BDA)
    x0 = k1 + jnp.zeros_like(p)
    x1 = p + k2
    inj = ((k2, ks2, 1), (ks2, k1, 2), (k1, k2, 3), (k2, ks2, 4), (ks2, k1, 5))
    for grp in range(5):
        for r in _ROTS[grp % 2]:
            x0 = x0 + x1
            x1 = ((x1 << np.int32(r)) |
                  lax.shift_right_logical(x1, np.int32(32 - r)))
            x1 = x1 ^ x0
        ka, kb, inc = inj[grp]
        x0 = x0 + ka
        x1 = x1 + kb + jnp.int32(inc)
    bits = x0 ^ x1
    fb = lax.shift_right_logical(bits, np.int32(9)) | jnp.int32(0x3F800000)
    f = lax.bitcast_convert_type(fb, jnp.float32) - np.float32(1.0)
    u = jnp.maximum(_TINY, f * (np.float32(1.0) - _TINY) + _TINY)
    return -jnp.log(-jnp.log(u))


BR = 8
TROW, TCOL = 8, 512
BLK = BR * BATCH


def _tc_gumbel_body(row0, kd_ref, g_ref):
    b = pl.program_id(0)
    k1 = kd_ref[0]
    k2 = kd_ref[1]
    iota2 = (lax.broadcasted_iota(jnp.int32, (TROW, TCOL), 0) * jnp.int32(TCOL)
             + lax.broadcasted_iota(jnp.int32, (TROW, TCOL), 1))
    base = row0 * jnp.int32(BATCH) + b * jnp.int32(BLK)

    def jstep(c, _):
        q0 = c * jnp.int32(TROW * TCOL)
        g = _threefry_gumbel(k1, k2, base + q0 + iota2)
        for r in range(TROW):
            g_ref[pl.ds(q0 + r * TCOL, TCOL)] = jnp.squeeze(
                lax.slice(g, (r, 0), (r + 1, TCOL)), axis=0)
        return _

    lax.fori_loop(0, BLK // (TROW * TCOL), jstep, 0)


def _make_tc_gumbel(row0):
    return pl.pallas_call(
        functools.partial(_tc_gumbel_body, row0),
        grid=(CHUNK_ROWS // BR,),
        in_specs=[pl.BlockSpec(memory_space=pltpu.SMEM)],
        out_specs=pl.BlockSpec((BLK,), lambda b: (b,)),
        out_shape=jax.ShapeDtypeStruct((CHUNK_ROWS * BATCH,), jnp.float32),
    )


def _sc_body(row0, state_hbm, g_hbm, t_hbm, out_hbm,
             state_v, rows_v, g_v, out_v, sem_r, sem_g):
    wid = lax.axis_index("s") * NC + lax.axis_index("c")
    lbase = wid * ROWS_PER_W
    pltpu.sync_copy(state_hbm, state_v)
    iota = lax.iota(jnp.int32, LANES)
    neg_inf = jnp.full((LANES,), -jnp.inf, jnp.float32)
    zero_i = jnp.zeros((LANES,), jnp.int32)
    big = jnp.int32(2 ** 30)

    acc = zero_i
    for grp in range(ROWS_PER_W // GROUP):
        lrow = lbase + grp * GROUP
        cp_r = pltpu.make_async_copy(
            t_hbm.at[pl.ds(row0 + lrow, GROUP), :], rows_v, sem_r)
        cp_g = pltpu.make_async_copy(
            g_hbm.at[pl.ds(lrow * BATCH, GROUP * BATCH)], g_v, sem_g)
        cp_r.start()
        cp_g.start()
        cp_r.wait()
        cp_g.wait()

        def inner(k, c):
            off = k * LANES
            idx = state_v[pl.ds(off, LANES)]
            j = off + iota
            new = []
            for r in range(GROUP):
                bv, bj = c[2 * r], c[2 * r + 1]
                val = (plsc.load_gather(rows_v,
                                        [jnp.full((LANES,), r, jnp.int32), idx])
                       + g_v[pl.ds(r * BATCH + off, LANES)])
                upd = val > bv
                new.append(jnp.where(upd, val, bv))
                new.append(jnp.where(upd, j, bj))
            return tuple(new)

        init = (neg_inf, zero_i) * GROUP
        res = lax.fori_loop(0, CHUNKS16, inner, init)
        for r in range(GROUP):
            bv, bj = res[2 * r], res[2 * r + 1]
            m = jnp.max(bv)
            mj = jnp.min(jnp.where(bv == m, bj, big))
            lane = (grp * GROUP + r) % LANES
            acc = jnp.where(iota == lane, mj, acc)
        if (grp * GROUP + GROUP) % LANES == 0:
            vec = ((grp * GROUP + GROUP) // LANES - 1) * LANES
            out_v[pl.ds(vec, LANES)] = acc

    pltpu.sync_copy(out_v, out_hbm.at[pl.ds(lbase, ROWS_PER_W)])


def _make_sc_chunk(row0):
    return pl.kernel(
        functools.partial(_sc_body, row0),
        out_type=jax.ShapeDtypeStruct((CHUNK_ROWS,), jnp.int32),
        mesh=plsc.VectorSubcoreMesh(core_axis_name="c", subcore_axis_name="s",
                                    num_cores=NC, num_subcores=NS),
        scratch_types=[
            pltpu.VMEM((BATCH,), jnp.int32),
            pltpu.VMEM((GROUP, N_STATES), jnp.float32),
            pltpu.VMEM((GROUP * BATCH,), jnp.float32),
            pltpu.VMEM((ROWS_PER_W,), jnp.int32),
            pltpu.SemaphoreType.DMA,
            pltpu.SemaphoreType.DMA,
        ],
        compiler_params=pltpu.CompilerParams(use_tc_tiling_on_sc=False,
                                             needs_layout_passes=False),
    )


_TC_CALLS = [_make_tc_gumbel(c * CHUNK_ROWS) for c in range(NB)]


@functools.lru_cache(maxsize=None)
def _sc_calls():
    return [_make_sc_chunk(c * CHUNK_ROWS) for c in range(NB)]


def kernel(state, rng, trans_matrix):
    kd = lax.bitcast_convert_type(jax.random.key_data(rng), jnp.int32)
    sc = _sc_calls()
    outs = []
    for c in range(NB):
        g_c = _TC_CALLS[c](kd)
        outs.append(sc[c](state, g_c, trans_matrix))
    return jnp.concatenate(outs)


# speedup vs baseline: 1.8566x; 1.4781x over previous
import functools

import jax
import jax.numpy as jnp
import numpy as np
from jax import lax
from jax.experimental import pallas as pl
from jax.experimental.pallas import tpu as pltpu
from jax.experimental.pallas import tpu_sc as plsc

N_STATES = 8192
BATCH = 4096
LANES = 16
NC, NS = 2, 16
NW = NC * NS
NB = 8
CHUNK_ROWS = N_STATES // NB
ROWS_PER_W = CHUNK_ROWS // NW
GROUP = 8
CHUNKS16 = BATCH // LANES

_TINY = np.float32(np.finfo(np.float32).tiny)


_ROTS = ((13, 15, 26, 6), (17, 29, 16, 24))


def _threefry_gumbel(k1, k2, p):
    ks2 = k1 ^ k2 ^ jnp.int32(0x1BD11BDA)
    x0 = k1 + jnp.zeros_like(p)
    x1 = p + k2
    inj = ((k2, ks2, 1), (ks2, k1, 2), (k1, k2, 3), (k2, ks2, 4), (ks2, k1, 5))
    for grp in range(5):
        for r in _ROTS[grp % 2]:
            x0 = x0 + x1
            x1 = ((x1 << np.int32(r)) |
                  lax.shift_right_logical(x1, np.int32(32 - r)))
            x1 = x1 ^ x0
        ka, kb, inc = inj[grp]
        x0 = x0 + ka
        x1 = x1 + kb + jnp.int32(inc)
    bits = x0 ^ x1
    fb = lax.shift_right_logical(bits, np.int32(9)) | jnp.int32(0x3F800000)
    f = lax.bitcast_convert_type(fb, jnp.float32) - np.float32(1.0)
    u = jnp.maximum(_TINY, f * (np.float32(1.0) - _TINY) + _TINY)
    return -jnp.log(-jnp.log(u))


BR = 8
TROW, TCOL = 8, 1024
BLK = BR * BATCH


def _tc_gumbel_body(row0, kd_ref, g_ref):
    b = pl.program_id(0)
    k1 = kd_ref[0]
    k2 = kd_ref[1]
    iota2 = (lax.broadcasted_iota(jnp.int32, (TROW, TCOL), 0) * jnp.int32(TCOL)
             + lax.broadcasted_iota(jnp.int32, (TROW, TCOL), 1))
    base = row0 * jnp.int32(BATCH) + b * jnp.int32(BLK)

    for c in range(BLK // (TROW * TCOL)):
        q0 = c * (TROW * TCOL)
        g = _threefry_gumbel(k1, k2, base + q0 + iota2)
        for r in range(TROW):
            g_ref[pl.ds(q0 + r * TCOL, TCOL)] = jnp.squeeze(
                lax.slice(g, (r, 0), (r + 1, TCOL)), axis=0)


def _make_tc_gumbel(row0):
    return pl.pallas_call(
        functools.partial(_tc_gumbel_body, row0),
        grid=(CHUNK_ROWS // BR,),
        in_specs=[pl.BlockSpec(memory_space=pltpu.SMEM)],
        out_specs=pl.BlockSpec((BLK,), lambda b: (b,)),
        out_shape=jax.ShapeDtypeStruct((CHUNK_ROWS * BATCH,), jnp.float32),
    )


def _sc_body(row0, state_hbm, g_hbm, t_hbm, out_hbm,
             state_v, rows_v, g_v, out_v, sem_r, sem_g):
    wid = lax.axis_index("s") * NC + lax.axis_index("c")
    lbase = wid * ROWS_PER_W
    pltpu.sync_copy(state_hbm, state_v)
    iota = lax.iota(jnp.int32, LANES)
    neg_inf = jnp.full((LANES,), -jnp.inf, jnp.float32)
    zero_i = jnp.zeros((LANES,), jnp.int32)
    big = jnp.int32(2 ** 30)

    acc = zero_i
    for grp in range(ROWS_PER_W // GROUP):
        lrow = lbase + grp * GROUP
        cp_r = pltpu.make_async_copy(
            t_hbm.at[pl.ds(row0 + lrow, GROUP), :], rows_v, sem_r)
        cp_g = pltpu.make_async_copy(
            g_hbm.at[pl.ds(lrow * BATCH, GROUP * BATCH)], g_v, sem_g)
        cp_r.start()
        cp_g.start()
        cp_r.wait()
        cp_g.wait()

        def inner(k, c):
            off = k * LANES
            idx = state_v[pl.ds(off, LANES)]
            j = off + iota
            new = []
            for r in range(GROUP):
                bv, bj = c[2 * r], c[2 * r + 1]
                val = (plsc.load_gather(rows_v,
                                        [jnp.full((LANES,), r, jnp.int32), idx])
                       + g_v[pl.ds(r * BATCH + off, LANES)])
                upd = val > bv
                new.append(jnp.where(upd, val, bv))
                new.append(jnp.where(upd, j, bj))
            return tuple(new)

        init = (neg_inf, zero_i) * GROUP
        res = lax.fori_loop(0, CHUNKS16, inner, init)
        for r in range(GROUP):
            bv, bj = res[2 * r], res[2 * r + 1]
            m = jnp.max(bv)
            mj = jnp.min(jnp.where(bv == m, bj, big))
            lane = (grp * GROUP + r) % LANES
            acc = jnp.where(iota == lane, mj, acc)
        if (grp * GROUP + GROUP) % LANES == 0:
            vec = ((grp * GROUP + GROUP) // LANES - 1) * LANES
            out_v[pl.ds(vec, LANES)] = acc

    pltpu.sync_copy(out_v, out_hbm.at[pl.ds(lbase, ROWS_PER_W)])


def _make_sc_chunk(row0):
    return pl.kernel(
        functools.partial(_sc_body, row0),
        out_type=jax.ShapeDtypeStruct((CHUNK_ROWS,), jnp.int32),
        mesh=plsc.VectorSubcoreMesh(core_axis_name="c", subcore_axis_name="s",
                                    num_cores=NC, num_subcores=NS),
        scratch_types=[
            pltpu.VMEM((BATCH,), jnp.int32),
            pltpu.VMEM((GROUP, N_STATES), jnp.float32),
            pltpu.VMEM((GROUP * BATCH,), jnp.float32),
            pltpu.VMEM((ROWS_PER_W,), jnp.int32),
            pltpu.SemaphoreType.DMA,
            pltpu.SemaphoreType.DMA,
        ],
        compiler_params=pltpu.CompilerParams(use_tc_tiling_on_sc=False,
                                             needs_layout_passes=False),
    )


_TC_CALLS = [_make_tc_gumbel(c * CHUNK_ROWS) for c in range(NB)]


@functools.lru_cache(maxsize=None)
def _sc_calls():
    return [_make_sc_chunk(c * CHUNK_ROWS) for c in range(NB)]


def kernel(state, rng, trans_matrix):
    kd = lax.bitcast_convert_type(jax.random.key_data(rng), jnp.int32)
    sc = _sc_calls()
    outs = []
    for c in range(NB):
        g_c = _TC_CALLS[c](kd)
        outs.append(sc[c](state, g_c, trans_matrix))
    return jnp.concatenate(outs)
